# baseline (speedup 1.0000x reference)
import functools

import numpy as np
import jax
import jax.numpy as jnp
from jax import lax
from jax.experimental import pallas as pl
from jax.experimental.pallas import tpu as pltpu
from jax.experimental.pallas import tpu_sc as plsc

N = 10000
E = 320000
D_IN = 128
HEADS = 8
DH = 16
AW = D_IN + 16

NC = 2
NS = 16
NW = NC * NS
EPW = E // NW
CH = 40
NCHUNK = EPW // CH
HCH = NCHUNK // 2
RPT = N // NS

_PERM = np.zeros(D_IN, np.int32)
for _p in range(4):
    for _i in range(16):
        _PERM[32 * _p + 2 * _i] = 32 * _p + _i
        _PERM[32 * _p + 2 * _i + 1] = 32 * _p + 16 + _i


def _qkv_body(x_ref, seed_ref, t_ref, d_ref, c_ref,
              wn_ref, ws_ref, wt_ref, wd_ref, wc_ref,
              wq_ref, wk_ref, wv_ref,
              h_ref, q_ref, kv_ref):
    h = ((((x_ref[...] * wn_ref[...]) + (seed_ref[...] * ws_ref[...]))
          + (t_ref[...] * wt_ref[...]))
         + (d_ref[...] * wd_ref[...])) + (c_ref[...] * wc_ref[...])
    h_ref[...] = h
    q = jnp.dot(h, wq_ref[...], preferred_element_type=jnp.float32) * 0.25
    q_ref[...] = q.astype(jnp.bfloat16)
    kv_ref[:, :D_IN] = jnp.dot(
        h, wk_ref[...], preferred_element_type=jnp.float32).astype(jnp.bfloat16)
    kv_ref[:, D_IN:] = jnp.dot(
        h, wv_ref[...], preferred_element_type=jnp.float32).astype(jnp.bfloat16)


_qkv_call = pl.pallas_call(
    _qkv_body,
    out_shape=[jax.ShapeDtypeStruct((N, D_IN), jnp.float32),
               jax.ShapeDtypeStruct((N, D_IN), jnp.bfloat16),
               jax.ShapeDtypeStruct((N, 2 * D_IN), jnp.bfloat16)],
)


def _edge_body(q_hbm, kv_hbm, src_hbm, dst_hbm, acc_out,
               srcb, dstb, qr0, kvr0, qr1, kvr1, wr0, wr1,
               acc_sh,
               sem_q0, sem_kv0, sem_q1, sem_kv1, sem_w0, sem_w1):
    cid = lax.axis_index("c")
    sid = lax.axis_index("s")
    lane = lax.iota(jnp.int32, 16)
    zvec = jnp.zeros((16,), jnp.float32)
    bufs = ((qr0, kvr0, wr0, sem_q0, sem_kv0, sem_w0),
            (qr1, kvr1, wr1, sem_q1, sem_kv1, sem_w1))

    def zrow(r, carry):
        for j in range(AW // 16):
            wr0[r, pl.ds(j * 16, 16)] = zvec
            wr1[r, pl.ds(j * 16, 16)] = zvec
        return carry

    lax.fori_loop(0, CH, zrow, 0)
    rb = sid * RPT
    nz = RPT // CH
    rem = RPT - nz * CH

    def zcopy(i, carry):
        pltpu.sync_copy(wr0, acc_sh.at[pl.ds(rb + i * CH, CH)])
        return carry

    lax.fori_loop(0, nz, zcopy, 0)
    pltpu.sync_copy(wr0.at[pl.ds(0, rem)],
                    acc_sh.at[pl.ds(rb + nz * CH, rem)])
    plsc.subcore_barrier()

    wrow = (cid * NS + sid) * (EPW // CH)
    PF = plsc.PackFormat.INTERLEAVED

    def start(g, b):
        qr, kvr, wr, sem_q, sem_kv, sem_w = bufs[b]
        pltpu.async_copy(q_hbm.at[dstb.at[g]], qr, sem_q)
        pltpu.async_copy(kv_hbm.at[srcb.at[g]], kvr, sem_kv)

    def compute(g, b):
        qr, kvr, wr, sem_q, sem_kv, sem_w = bufs[b]
        pltpu.make_async_copy(q_hbm.at[dstb.at[g]], qr, sem_q).wait()
        pltpu.make_async_copy(kv_hbm.at[srcb.at[g]], kvr, sem_kv).wait()
        pltpu.make_async_copy(wr, acc_sh.at[dstb.at[g]], sem_w).wait()

        @plsc.parallel_loop(0, CH, unroll=6)
        def edge(e):
            terms = []
            for p in range(4):
                qa, qb = plsc.unpack(qr[e, pl.ds(32 * p, 32)], format=PF,
                                     preferred_element_type=jnp.float32)
                ka, kb = plsc.unpack(kvr[e, pl.ds(32 * p, 32)], format=PF,
                                     preferred_element_type=jnp.float32)
                terms.append(jnp.where(lane == 2 * p, jnp.sum(qa * ka), 0.0))
                terms.append(jnp.where(lane == 2 * p + 1,
                                       jnp.sum(qb * kb), 0.0))
            sv = (((terms[0] + terms[1]) + (terms[2] + terms[3]))
                  + ((terms[4] + terms[5]) + (terms[6] + terms[7])))
            ex = jnp.exp(sv)
            wr[e, pl.ds(D_IN, 16)] = ex
            for p in range(4):
                va, vb = plsc.unpack(kvr[e, pl.ds(D_IN + 32 * p, 32)],
                                     format=PF,
                                     preferred_element_type=jnp.float32)
                wa = ex.at[jnp.full((16,), 2 * p, jnp.int32)].get(
                    mode="promise_in_bounds")
                wb = ex.at[jnp.full((16,), 2 * p + 1, jnp.int32)].get(
                    mode="promise_in_bounds")
                wr[e, pl.ds(32 * p, 16)] = va * wa
                wr[e, pl.ds(32 * p + 16, 16)] = vb * wb

        pltpu.async_copy(wr, acc_sh.at[dstb.at[g]], sem_w, add=True)

    for half in range(NCHUNK // HCH):
        pltpu.sync_copy(src_hbm.at[pl.ds(wrow + half * HCH, HCH)], srcb)
        pltpu.sync_copy(dst_hbm.at[pl.ds(wrow + half * HCH, HCH)], dstb)
        if half == 0:
            pltpu.async_copy(wr0, acc_sh.at[dstb.at[0]], sem_w0, add=True)
            pltpu.async_copy(wr1, acc_sh.at[dstb.at[1]], sem_w1, add=True)
        start(0, 0)
        start(1, 1)

        def pair(i2, carry):
            g = i2 * 2
            compute(g, 0)
            start(g + 2, 0)
            compute(g + 1, 1)
            start(g + 3, 1)
            return carry

        lax.fori_loop(0, (HCH - 3) // 2, pair, 0)
        compute(HCH - 3, 0)
        start(HCH - 1, 0)
        compute(HCH - 2, 1)
        compute(HCH - 1, 0)
    pltpu.make_async_copy(wr0, acc_sh.at[dstb.at[HCH - 1]], sem_w0).wait()
    pltpu.make_async_copy(wr1, acc_sh.at[dstb.at[HCH - 2]], sem_w1).wait()
    plsc.subcore_barrier()

    pltpu.sync_copy(acc_sh.at[pl.ds(rb, RPT)], acc_out.at[cid, pl.ds(rb, RPT)])


_edge_call = functools.partial(
    pl.kernel,
    out_type=jax.ShapeDtypeStruct((NC, N, AW), jnp.float32),
    mesh=plsc.VectorSubcoreMesh(core_axis_name="c", subcore_axis_name="s"),
    compiler_params=pltpu.CompilerParams(use_tc_tiling_on_sc=False,
                                         needs_layout_passes=False),
    scratch_types=(
        [pltpu.VMEM((HCH, CH), jnp.int32),
         pltpu.VMEM((HCH, CH), jnp.int32)]
        + [pltpu.VMEM((CH, D_IN), jnp.bfloat16),
           pltpu.VMEM((CH, 2 * D_IN), jnp.bfloat16)] * 2
        + [pltpu.VMEM((CH, AW), jnp.float32)] * 2
        + [pltpu.VMEM_SHARED((N, AW), jnp.float32)]
        + [pltpu.SemaphoreType.DMA] * 6
    ),
)(_edge_body)


def _head_body(a0_ref, a1_ref, h_ref,
               wo_ref, w1_ref, b1_ref, w2_ref, b2_ref, bmat_ref, out_ref):
    aggs = a0_ref[:, :D_IN] + a1_ref[:, :D_IN]
    dens = a0_ref[:, D_IN:] + a1_ref[:, D_IN:]
    rec = 1.0 / (dens + 1e-16)
    rec128 = jnp.dot(rec, bmat_ref[...], preferred_element_type=jnp.float32,
        precision=lax.Precision.HIGHEST)
    attn = aggs * rec128
    out = jnp.dot(attn, wo_ref[...], preferred_element_type=jnp.float32)
    out = out + h_ref[...]
    hm = jnp.maximum(
        jnp.dot(out, w1_ref[...], preferred_element_type=jnp.float32)
        + b1_ref[...], 0.0)
    out_ref[...] = (jnp.dot(hm, w2_ref[...], preferred_element_type=jnp.float32)
                    + b2_ref[...])


_head_call = pl.pallas_call(
    _head_body,
    out_shape=jax.ShapeDtypeStruct((N, 1), jnp.float32),
)


def kernel(X, Seed, T, D, C, edge_index,
           W_node, W_seed, W_time, W_deg, W_clu,
           Wq, Wk, Wv, Wo, W1, b1, W2, b2):
    h, q, kv = _qkv_call(X, Seed, T, D, C,
                         W_node, W_seed, W_time, W_deg, W_clu,
                         Wq[:, _PERM], Wk[:, _PERM], Wv[:, _PERM])

    src = edge_index[0].reshape(E // CH, CH)
    dst = edge_index[1].reshape(E // CH, CH)
    acc = _edge_call(q, kv, src, dst)

    eye = jnp.eye(16, dtype=jnp.float32)[:, :HEADS]
    bmat = jnp.repeat(eye, DH, axis=1)

    predX = _head_call(acc[0], acc[1], h,
                       Wo, W1, b1.reshape(1, D_IN), W2,
                       b2.reshape(1, 1), bmat)
    return predX


# speedup vs baseline: 1.0630x; 1.0630x over previous
import functools

import numpy as np
import jax
import jax.numpy as jnp
from jax import lax
from jax.experimental import pallas as pl
from jax.experimental.pallas import tpu as pltpu
from jax.experimental.pallas import tpu_sc as plsc

N = 10000
E = 320000
D_IN = 128
HEADS = 8
DH = 16
AW = D_IN + 16

NC = 2
NS = 16
NW = NC * NS
EPW = E // NW
CH = 40
NCHUNK = EPW // CH
HCH = NCHUNK // 2
RPT = N // NS

_PERM = np.zeros(D_IN, np.int32)
for _p in range(4):
    for _i in range(16):
        _PERM[32 * _p + 2 * _i] = 32 * _p + _i
        _PERM[32 * _p + 2 * _i + 1] = 32 * _p + 16 + _i


def _qkv_body(x_ref, seed_ref, t_ref, d_ref, c_ref,
              wn_ref, ws_ref, wt_ref, wd_ref, wc_ref,
              wq_ref, wk_ref, wv_ref,
              h_ref, q_ref, kv_ref):
    h = ((((x_ref[...] * wn_ref[...]) + (seed_ref[...] * ws_ref[...]))
          + (t_ref[...] * wt_ref[...]))
         + (d_ref[...] * wd_ref[...])) + (c_ref[...] * wc_ref[...])
    h_ref[...] = h
    q = jnp.dot(h, wq_ref[...], preferred_element_type=jnp.float32) * 0.25
    q_ref[...] = q.astype(jnp.bfloat16)
    kv_ref[:, :D_IN] = jnp.dot(
        h, wk_ref[...], preferred_element_type=jnp.float32).astype(jnp.bfloat16)
    kv_ref[:, D_IN:] = jnp.dot(
        h, wv_ref[...], preferred_element_type=jnp.float32).astype(jnp.bfloat16)


_qkv_call = pl.pallas_call(
    _qkv_body,
    out_shape=[jax.ShapeDtypeStruct((N, D_IN), jnp.float32),
               jax.ShapeDtypeStruct((N, D_IN), jnp.bfloat16),
               jax.ShapeDtypeStruct((N, 2 * D_IN), jnp.bfloat16)],
)


def _edge_body(q_hbm, kv_hbm, src_hbm, dst_hbm, acc_out,
               srcb, dstb, qr0, kvr0, qr1, kvr1, wr0, wr1,
               acc_sh,
               sem_q0, sem_kv0, sem_q1, sem_kv1, sem_w0, sem_w1):
    cid = lax.axis_index("c")
    sid = lax.axis_index("s")
    lane = lax.iota(jnp.int32, 16)
    zvec = jnp.zeros((16,), jnp.float32)
    bufs = ((qr0, kvr0, wr0, sem_q0, sem_kv0, sem_w0),
            (qr1, kvr1, wr1, sem_q1, sem_kv1, sem_w1))

    def zrow(r, carry):
        for j in range(AW // 16):
            wr0[r, pl.ds(j * 16, 16)] = zvec
            wr1[r, pl.ds(j * 16, 16)] = zvec
        return carry

    lax.fori_loop(0, CH, zrow, 0)
    rb = sid * RPT
    nz = RPT // CH
    rem = RPT - nz * CH

    def zcopy(i, carry):
        pltpu.sync_copy(wr0, acc_sh.at[pl.ds(rb + i * CH, CH)])
        return carry

    lax.fori_loop(0, nz, zcopy, 0)
    pltpu.sync_copy(wr0.at[pl.ds(0, rem)],
                    acc_sh.at[pl.ds(rb + nz * CH, rem)])
    plsc.subcore_barrier()

    wrow = (cid * NS + sid) * (EPW // CH)
    PF = plsc.PackFormat.INTERLEAVED

    def start(g, b):
        qr, kvr, wr, sem_q, sem_kv, sem_w = bufs[b]
        pltpu.async_copy(q_hbm.at[dstb.at[g]], qr, sem_q)
        pltpu.async_copy(kv_hbm.at[srcb.at[g]], kvr, sem_kv)

    def compute(g, b):
        qr, kvr, wr, sem_q, sem_kv, sem_w = bufs[b]
        pltpu.make_async_copy(q_hbm.at[dstb.at[g]], qr, sem_q).wait()
        pltpu.make_async_copy(kv_hbm.at[srcb.at[g]], kvr, sem_kv).wait()
        pltpu.make_async_copy(wr, acc_sh.at[dstb.at[g]], sem_w).wait()

        @plsc.parallel_loop(0, CH, unroll=4)
        def edge(e):
            terms = []
            for p in range(4):
                qa, qb = plsc.unpack(qr[e, pl.ds(32 * p, 32)], format=PF,
                                     preferred_element_type=jnp.float32)
                ka, kb = plsc.unpack(kvr[e, pl.ds(32 * p, 32)], format=PF,
                                     preferred_element_type=jnp.float32)
                terms.append(jnp.where(lane == 2 * p, jnp.sum(qa * ka), 0.0))
                terms.append(jnp.where(lane == 2 * p + 1,
                                       jnp.sum(qb * kb), 0.0))
            sv = (((terms[0] + terms[1]) + (terms[2] + terms[3]))
                  + ((terms[4] + terms[5]) + (terms[6] + terms[7])))
            ex = jnp.exp(sv)
            wr[e, pl.ds(D_IN, 16)] = ex
            for p in range(4):
                va, vb = plsc.unpack(kvr[e, pl.ds(D_IN + 32 * p, 32)],
                                     format=PF,
                                     preferred_element_type=jnp.float32)
                wa = ex.at[jnp.full((16,), 2 * p, jnp.int32)].get(
                    mode="promise_in_bounds")
                wb = ex.at[jnp.full((16,), 2 * p + 1, jnp.int32)].get(
                    mode="promise_in_bounds")
                wr[e, pl.ds(32 * p, 16)] = va * wa
                wr[e, pl.ds(32 * p + 16, 16)] = vb * wb

        pltpu.async_copy(wr, acc_sh.at[dstb.at[g]], sem_w, add=True)

    for half in range(NCHUNK // HCH):
        pltpu.sync_copy(src_hbm.at[pl.ds(wrow + half * HCH, HCH)], srcb)
        pltpu.sync_copy(dst_hbm.at[pl.ds(wrow + half * HCH, HCH)], dstb)
        if half == 0:
            pltpu.async_copy(wr0, acc_sh.at[dstb.at[0]], sem_w0, add=True)
            pltpu.async_copy(wr1, acc_sh.at[dstb.at[1]], sem_w1, add=True)
        start(0, 0)
        start(1, 1)

        def pair(i2, carry):
            g = i2 * 2
            compute(g, 0)
            start(g + 2, 0)
            compute(g + 1, 1)
            start(g + 3, 1)
            return carry

        lax.fori_loop(0, (HCH - 3) // 2, pair, 0)
        compute(HCH - 3, 0)
        start(HCH - 1, 0)
        compute(HCH - 2, 1)
        compute(HCH - 1, 0)
    pltpu.make_async_copy(wr0, acc_sh.at[dstb.at[HCH - 1]], sem_w0).wait()
    pltpu.make_async_copy(wr1, acc_sh.at[dstb.at[HCH - 2]], sem_w1).wait()
    plsc.subcore_barrier()

    pltpu.sync_copy(acc_sh.at[pl.ds(rb, RPT)], acc_out.at[cid, pl.ds(rb, RPT)])


_edge_call = functools.partial(
    pl.kernel,
    out_type=jax.ShapeDtypeStruct((NC, N, AW), jnp.float32),
    mesh=plsc.VectorSubcoreMesh(core_axis_name="c", subcore_axis_name="s"),
    compiler_params=pltpu.CompilerParams(use_tc_tiling_on_sc=False,
                                         needs_layout_passes=False),
    scratch_types=(
        [pltpu.VMEM((HCH, CH), jnp.int32),
         pltpu.VMEM((HCH, CH), jnp.int32)]
        + [pltpu.VMEM((CH, D_IN), jnp.bfloat16),
           pltpu.VMEM((CH, 2 * D_IN), jnp.bfloat16)] * 2
        + [pltpu.VMEM((CH, AW), jnp.float32)] * 2
        + [pltpu.VMEM_SHARED((N, AW), jnp.float32)]
        + [pltpu.SemaphoreType.DMA] * 6
    ),
)(_edge_body)


def _head_body(a0_ref, a1_ref, h_ref,
               wo_ref, w1_ref, b1_ref, w2_ref, b2_ref, bmat_ref, out_ref):
    aggs = a0_ref[:, :D_IN] + a1_ref[:, :D_IN]
    dens = a0_ref[:, D_IN:] + a1_ref[:, D_IN:]
    rec = 1.0 / (dens + 1e-16)
    rec128 = jnp.dot(rec, bmat_ref[...], preferred_element_type=jnp.float32,
        precision=lax.Precision.HIGHEST)
    attn = aggs * rec128
    out = jnp.dot(attn, wo_ref[...], preferred_element_type=jnp.float32)
    out = out + h_ref[...]
    hm = jnp.maximum(
        jnp.dot(out, w1_ref[...], preferred_element_type=jnp.float32)
        + b1_ref[...], 0.0)
    out_ref[...] = (jnp.dot(hm, w2_ref[...], preferred_element_type=jnp.float32)
                    + b2_ref[...])


_head_call = pl.pallas_call(
    _head_body,
    out_shape=jax.ShapeDtypeStruct((N, 1), jnp.float32),
)


def kernel(X, Seed, T, D, C, edge_index,
           W_node, W_seed, W_time, W_deg, W_clu,
           Wq, Wk, Wv, Wo, W1, b1, W2, b2):
    h, q, kv = _qkv_call(X, Seed, T, D, C,
                         W_node, W_seed, W_time, W_deg, W_clu,
                         Wq[:, _PERM], Wk[:, _PERM], Wv[:, _PERM])

    src = edge_index[0].reshape(E // CH, CH)
    dst = edge_index[1].reshape(E // CH, CH)
    acc = _edge_call(q, kv, src, dst)

    eye = jnp.eye(16, dtype=jnp.float32)[:, :HEADS]
    bmat = jnp.repeat(eye, DH, axis=1)

    predX = _head_call(acc[0], acc[1], h,
                       Wo, W1, b1.reshape(1, D_IN), W2,
                       b2.reshape(1, 1), bmat)
    return predX
